# Initial kernel scaffold; baseline (speedup 1.0000x reference)
import jax
import jax.numpy as jnp
from jax.experimental import pallas as pl


def kernel(x, aug_table, mag_table, W0, b0, W1, b1, W2, b2, Wout, bout):
    raise NotImplementedError("write your pallas kernel here")


# speedup vs baseline: 1.1552x; 1.1552x over previous
import functools

import jax
import jax.numpy as jnp
from jax import lax
from jax.experimental import pallas as pl
from jax.experimental.pallas import tpu as pltpu
from jax.experimental.pallas import tpu_sc as plsc

B = 16384
EMBED_DIM = 128
HIDDEN = 2048

NUM_CORES = 2
NUM_SUBCORES = 16
NW = NUM_CORES * NUM_SUBCORES

BG = 4 * B
BPW = BG // NW
CHUNK = 256
NCH = BPW // CHUNK


def _sc_gather(table, idx):
    mesh = plsc.VectorSubcoreMesh(core_axis_name="c", subcore_axis_name="s")

    @functools.partial(
        pl.kernel,
        mesh=mesh,
        out_type=jax.ShapeDtypeStruct((BG, EMBED_DIM), jnp.float32),
        scratch_types=[
            pltpu.VMEM((BPW,), jnp.int32),
            pltpu.VMEM((CHUNK, EMBED_DIM), jnp.float32),
            pltpu.VMEM((CHUNK, EMBED_DIM), jnp.float32),
            pltpu.SemaphoreType.DMA,
            pltpu.SemaphoreType.DMA,
        ],
    )
    def k(table_hbm, idx_hbm, out_hbm, idx_v, rows0, rows1, sem0, sem1):
        wid = lax.axis_index("s") * NUM_CORES + lax.axis_index("c")
        base = wid * BPW
        pltpu.sync_copy(idx_hbm.at[pl.ds(base, BPW)], idx_v)
        bufs = (rows0, rows1)
        sems = (sem0, sem1)
        copies = [None, None]
        for c in range(NCH):
            s = c % 2
            copies[s] = pltpu.async_copy(
                table_hbm.at[idx_v.at[pl.ds(c * CHUNK, CHUNK)]], bufs[s], sems[s]
            )
            if c >= 1:
                p = (c - 1) % 2
                copies[p].wait()
                pltpu.sync_copy(bufs[p], out_hbm.at[pl.ds(base + (c - 1) * CHUNK, CHUNK)])
        last = (NCH - 1) % 2
        copies[last].wait()
        pltpu.sync_copy(bufs[last], out_hbm.at[pl.ds(base + (NCH - 1) * CHUNK, CHUNK)])

    return k(table, idx)


TB = 512


def _mlp_body(g_ref, w0_ref, b0_ref, w1_ref, b1_ref, w2_ref, b2_ref,
              wout_ref, bout_ref, y_ref):
    h = jnp.dot(g_ref[...], w0_ref[...], preferred_element_type=jnp.float32)
    h = jnp.maximum(h + b0_ref[...], 0.0)
    h = jnp.dot(h, w1_ref[...], preferred_element_type=jnp.float32)
    h = jnp.maximum(h + b1_ref[...], 0.0)
    h = jnp.dot(h, w2_ref[...], preferred_element_type=jnp.float32)
    h = jnp.maximum(h + b2_ref[...], 0.0)
    y = jnp.dot(h, wout_ref[...], preferred_element_type=jnp.float32)
    y_ref[...] = y + bout_ref[...]


def _mlp(g, W0, b0, W1, b1, W2, b2, Wout, bout):
    nb = B // TB
    full = lambda shape: pl.BlockSpec(shape, lambda i: (0, 0))
    return pl.pallas_call(
        _mlp_body,
        grid=(nb,),
        in_specs=[
            pl.BlockSpec((TB, 4 * EMBED_DIM), lambda i: (i, 0)),
            full((4 * EMBED_DIM, HIDDEN)),
            full((1, HIDDEN)),
            full((HIDDEN, HIDDEN)),
            full((1, HIDDEN)),
            full((HIDDEN, HIDDEN)),
            full((1, HIDDEN)),
            full((HIDDEN, 1)),
            full((1, 1)),
        ],
        out_specs=pl.BlockSpec((TB, 1), lambda i: (i, 0)),
        out_shape=jax.ShapeDtypeStruct((B, 1), jnp.float32),
        compiler_params=pltpu.CompilerParams(
            dimension_semantics=("arbitrary",),
        ),
    )(g, W0, b0.reshape(1, HIDDEN), W1, b1.reshape(1, HIDDEN),
      W2, b2.reshape(1, HIDDEN), Wout, bout.reshape(1, 1))


def kernel(x, aug_table, mag_table, W0, b0, W1, b1, W2, b2, Wout, bout):
    del mag_table
    idx = x.reshape(-1).astype(jnp.int32)
    g = _sc_gather(aug_table, idx)
    g = g.reshape(B, 4 * EMBED_DIM)
    return _mlp(g, W0, b0, W1, b1, W2, b2, Wout, bout)
